# baseline (speedup 1.0000x reference)
import jax
import jax.numpy as jnp
from jax import lax
from jax.experimental import pallas as pl
from jax.experimental.pallas import tpu as pltpu
from jax.experimental.pallas import tpu_sc as plsc

NC = 2
NS = 16
LANES = 16

B = 4
T = 2048
D = 128
NW = NC * NS
TPW = T // NW
NQ = 4
Q = TPW // NQ
WAVES = ((0, 1), (2, 3))


def _body(tok_hbm, x_hbm, pos_hbm, out_hbm,
          idx_v, tok_v, pos_v, pos_sem, idx_sem, g_sems, st_sem):
    wid = lax.axis_index("s") * NC + lax.axis_index("c")
    p0 = wid * TPW

    with jax.named_scope("pos_idx"):
        pos_cp = pltpu.async_copy(pos_hbm.at[pl.ds(p0, TPW)], pos_v, pos_sem)
        icps = [
            pltpu.async_copy(x_hbm.at[b, pl.ds(p0, TPW)], idx_v.at[b], idx_sem)
            for b in range(B)
        ]
        for cp in icps:
            cp.wait()

    def fire_wave(bs):
        return [
            pltpu.async_copy(tok_hbm.at[idx_v.at[b]], tok_v.at[b], g_sems[b])
            for b in bs
        ]

    with jax.named_scope("gather_fire_a"):
        gcps_a = fire_wave(WAVES[0])
    with jax.named_scope("pos_wait"):
        pos_cp.wait()

    st_cps = []

    def add_store_wave(bs):
        for q in range(NQ):
            def add_row(t, carry, q=q, bs=bs):
                t0 = q * Q + t
                for j in range(D // LANES):
                    sl = pl.ds(j * LANES, LANES)
                    p = pos_v[t0, sl]
                    for b in bs:
                        tok_v[b, t0, sl] = tok_v[b, t0, sl] + p
                return carry

            with jax.named_scope("add_loop"):
                lax.fori_loop(0, Q, add_row, 0)
            with jax.named_scope("store_fire"):
                for b in bs:
                    st_cps.append(
                        pltpu.async_copy(
                            tok_v.at[b, pl.ds(q * Q, Q)],
                            out_hbm.at[b, pl.ds(p0 + q * Q, Q)],
                            st_sem,
                        )
                    )

    with jax.named_scope("gather_wait_a"):
        for cp in gcps_a:
            cp.wait()
    with jax.named_scope("gather_fire_b"):
        gcps_b = fire_wave(WAVES[1])

    add_store_wave(WAVES[0])

    with jax.named_scope("gather_wait_b"):
        for cp in gcps_b:
            cp.wait()

    add_store_wave(WAVES[1])

    with jax.named_scope("store_drain"):
        for cp in st_cps:
            cp.wait()


@jax.jit
def kernel(x, tok_table, pos_table):
    mesh = plsc.VectorSubcoreMesh(
        core_axis_name="c", subcore_axis_name="s",
        num_cores=NC, num_subcores=NS,
    )
    run = pl.kernel(
        _body,
        out_type=jax.ShapeDtypeStruct((B, T, D), jnp.float32),
        mesh=mesh,
        scratch_types=[
            pltpu.VMEM((B, TPW), jnp.int32),
            pltpu.VMEM((B, TPW, D), jnp.float32),
            pltpu.VMEM((TPW, D), jnp.float32),
            pltpu.SemaphoreType.DMA,
            pltpu.SemaphoreType.DMA,
            [pltpu.SemaphoreType.DMA] * B,
            pltpu.SemaphoreType.DMA,
        ],
    )
    return run(tok_table, x, pos_table)


# speedup vs baseline: 1.0189x; 1.0189x over previous
import jax
import jax.numpy as jnp
from jax import lax
from jax.experimental import pallas as pl
from jax.experimental.pallas import tpu as pltpu
from jax.experimental.pallas import tpu_sc as plsc

NC = 2
NS = 16
LANES = 16

B = 4
T = 2048
D = 128
NW = NC * NS
TPW = T // NW
NQ = 4
Q = TPW // NQ
DEPTH = 2


def _body(tok_hbm, x_hbm, pos_hbm, out_hbm,
          idx_v, tok_v, pos_v, pos_sem, idx_sem, g_sems, st_sem):
    wid = lax.axis_index("s") * NC + lax.axis_index("c")
    p0 = wid * TPW

    with jax.named_scope("pos_idx"):
        pos_cp = pltpu.async_copy(pos_hbm.at[pl.ds(p0, TPW)], pos_v, pos_sem)
        icps = [
            pltpu.async_copy(x_hbm.at[b, pl.ds(p0, TPW)], idx_v.at[b], idx_sem)
            for b in range(B)
        ]
        for cp in icps:
            cp.wait()

    def fire_q(q):
        return [
            pltpu.async_copy(
                tok_hbm.at[idx_v.at[b, pl.ds(q * Q, Q)]],
                tok_v.at[b, pl.ds(q * Q, Q)],
                g_sems[b * NQ + q],
            )
            for b in range(B)
        ]

    gcps = {}
    with jax.named_scope("gather_fire"):
        for q in range(DEPTH):
            gcps[q] = fire_q(q)
    with jax.named_scope("pos_wait"):
        pos_cp.wait()

    st_cps = []
    for q in range(NQ):
        with jax.named_scope("gather_wait"):
            for cp in gcps[q]:
                cp.wait()
        if q + DEPTH < NQ:
            with jax.named_scope("gather_fire_next"):
                gcps[q + DEPTH] = fire_q(q + DEPTH)

        def add_row(t, carry, q=q):
            t0 = q * Q + t
            for j in range(D // LANES):
                sl = pl.ds(j * LANES, LANES)
                p = pos_v[t0, sl]
                for b in range(B):
                    tok_v[b, t0, sl] = tok_v[b, t0, sl] + p
            return carry

        with jax.named_scope("add_loop"):
            lax.fori_loop(0, Q, add_row, 0)
        with jax.named_scope("store_fire"):
            for b in range(B):
                st_cps.append(
                    pltpu.async_copy(
                        tok_v.at[b, pl.ds(q * Q, Q)],
                        out_hbm.at[b, pl.ds(p0 + q * Q, Q)],
                        st_sem,
                    )
                )

    with jax.named_scope("store_drain"):
        for cp in st_cps:
            cp.wait()


@jax.jit
def kernel(x, tok_table, pos_table):
    mesh = plsc.VectorSubcoreMesh(
        core_axis_name="c", subcore_axis_name="s",
        num_cores=NC, num_subcores=NS,
    )
    run = pl.kernel(
        _body,
        out_type=jax.ShapeDtypeStruct((B, T, D), jnp.float32),
        mesh=mesh,
        scratch_types=[
            pltpu.VMEM((B, TPW), jnp.int32),
            pltpu.VMEM((B, TPW, D), jnp.float32),
            pltpu.VMEM((TPW, D), jnp.float32),
            pltpu.SemaphoreType.DMA,
            pltpu.SemaphoreType.DMA,
            [pltpu.SemaphoreType.DMA] * (B * NQ),
            pltpu.SemaphoreType.DMA,
        ],
    )
    return run(tok_table, x, pos_table)


# speedup vs baseline: 1.0203x; 1.0014x over previous
import jax
import jax.numpy as jnp
from jax import lax
from jax.experimental import pallas as pl
from jax.experimental.pallas import tpu as pltpu
from jax.experimental.pallas import tpu_sc as plsc

NC = 2
NS = 16
LANES = 16

B = 4
T = 2048
D = 128
NW = NC * NS
TPW = T // NW
NQ = 4
Q = TPW // NQ
DEPTH = 3


def _body(tok_hbm, x_hbm, pos_hbm, out_hbm,
          idx_v, tok_v, pos_v, pos_sem, idx_sem, g_sems, st_sem):
    wid = lax.axis_index("s") * NC + lax.axis_index("c")
    p0 = wid * TPW

    with jax.named_scope("pos_idx"):
        pos_cp = pltpu.async_copy(pos_hbm.at[pl.ds(p0, TPW)], pos_v, pos_sem)
        icps = [
            pltpu.async_copy(x_hbm.at[b, pl.ds(p0, TPW)], idx_v.at[b], idx_sem)
            for b in range(B)
        ]
        for cp in icps:
            cp.wait()

    def fire_q(q):
        return [
            pltpu.async_copy(
                tok_hbm.at[idx_v.at[b, pl.ds(q * Q, Q)]],
                tok_v.at[b, pl.ds(q * Q, Q)],
                g_sems[b * NQ + q],
            )
            for b in range(B)
        ]

    gcps = {}
    with jax.named_scope("gather_fire"):
        for q in range(DEPTH):
            gcps[q] = fire_q(q)
    with jax.named_scope("pos_wait"):
        pos_cp.wait()

    st_cps = []
    for q in range(NQ):
        with jax.named_scope("gather_wait"):
            for cp in gcps[q]:
                cp.wait()
        if q + DEPTH < NQ:
            with jax.named_scope("gather_fire_next"):
                gcps[q + DEPTH] = fire_q(q + DEPTH)

        def add_row(t, carry, q=q):
            t0 = q * Q + t
            for j in range(D // LANES):
                sl = pl.ds(j * LANES, LANES)
                p = pos_v[t0, sl]
                for b in range(B):
                    tok_v[b, t0, sl] = tok_v[b, t0, sl] + p
            return carry

        with jax.named_scope("add_loop"):
            lax.fori_loop(0, Q, add_row, 0)
        with jax.named_scope("store_fire"):
            for b in range(B):
                st_cps.append(
                    pltpu.async_copy(
                        tok_v.at[b, pl.ds(q * Q, Q)],
                        out_hbm.at[b, pl.ds(p0 + q * Q, Q)],
                        st_sem,
                    )
                )

    with jax.named_scope("store_drain"):
        for cp in st_cps:
            cp.wait()


@jax.jit
def kernel(x, tok_table, pos_table):
    mesh = plsc.VectorSubcoreMesh(
        core_axis_name="c", subcore_axis_name="s",
        num_cores=NC, num_subcores=NS,
    )
    run = pl.kernel(
        _body,
        out_type=jax.ShapeDtypeStruct((B, T, D), jnp.float32),
        mesh=mesh,
        scratch_types=[
            pltpu.VMEM((B, TPW), jnp.int32),
            pltpu.VMEM((B, TPW, D), jnp.float32),
            pltpu.VMEM((TPW, D), jnp.float32),
            pltpu.SemaphoreType.DMA,
            pltpu.SemaphoreType.DMA,
            [pltpu.SemaphoreType.DMA] * (B * NQ),
            pltpu.SemaphoreType.DMA,
        ],
    )
    return run(tok_table, x, pos_table)
